# baseline (speedup 1.0000x reference)
import jax
import jax.numpy as jnp
from jax.experimental import pallas as pl


def _body(ew_ref, pos_ref, out_ref):
    base = ew_ref[...] + pos_ref[...]
    out_ref[...] = jnp.broadcast_to(base[None], out_ref.shape)


def kernel(x, embed_weight, pos):
    b = x.shape[0]
    n, d = embed_weight.shape
    bb = 8
    grid = (b // bb,)
    return pl.pallas_call(
        _body,
        grid=grid,
        in_specs=[
            pl.BlockSpec((n, d), lambda i: (0, 0)),
            pl.BlockSpec((n, d), lambda i: (0, 0)),
        ],
        out_specs=pl.BlockSpec((bb, n, d), lambda i: (i, 0, 0)),
        out_shape=jax.ShapeDtypeStruct((b, n, d), jnp.float32),
    )(embed_weight, pos)


# speedup vs baseline: 1.2882x; 1.2882x over previous
import jax
import jax.numpy as jnp
from jax.experimental import pallas as pl
from jax.experimental.pallas import tpu as pltpu

REP = 16


def _body(ew_ref, pos_ref, out_ref, scratch, sem):
    base = ew_ref[...] + pos_ref[...]
    for r in range(REP):
        scratch[r] = base
    b = out_ref.shape[0]
    copies = [
        pltpu.make_async_copy(scratch, out_ref.at[pl.ds(i * REP, REP)], sem)
        for i in range(b // REP)
    ]
    for c in copies:
        c.start()
    for c in copies:
        c.wait()


def kernel(x, embed_weight, pos):
    b = x.shape[0]
    n, d = embed_weight.shape
    return pl.pallas_call(
        _body,
        in_specs=[
            pl.BlockSpec(memory_space=pltpu.VMEM),
            pl.BlockSpec(memory_space=pltpu.VMEM),
        ],
        out_specs=pl.BlockSpec(memory_space=pl.ANY),
        out_shape=jax.ShapeDtypeStruct((b, n, d), jnp.float32),
        scratch_shapes=[
            pltpu.VMEM((REP, n, d), jnp.float32),
            pltpu.SemaphoreType.DMA,
        ],
    )(embed_weight, pos)
